# Initial kernel scaffold; baseline (speedup 1.0000x reference)
import functools

import jax
import jax.numpy as jnp
from jax import lax
from jax.experimental import pallas as pl
from jax.experimental.pallas import tpu as pltpu
from jax.experimental.pallas import tpu_sc as plsc

N = 10000
NPAD = 10240
E = 160000
NC, NS, L = 2, 16, 16
NW = NC * NS
CHUNK = 128
EPT = 5120
EPAD = EPT * NW
NCHUNK = EPT // CHUNK
RPS = NPAD // NS

_mesh = plsc.VectorSubcoreMesh(core_axis_name="c", subcore_axis_name="s")


def _zero_rows(stage_v):
    z = jnp.zeros((L,), jnp.float32)

    def body(i, _):
        stage_v[i, :] = z
        return 0

    lax.fori_loop(0, RPS, body, 0)


def _my_wid():
    return lax.axis_index("s") * NC + lax.axis_index("c")


def _acc_epilogue(acc_sh, stage_v, out_hbm, cid, sid):
    plsc.subcore_barrier()
    pltpu.sync_copy(acc_sh.at[pl.ds(sid * RPS, RPS)], stage_v)
    pltpu.sync_copy(stage_v, out_hbm.at[cid].at[pl.ds(sid * RPS, RPS)])


@functools.partial(
    pl.kernel,
    out_type=jax.ShapeDtypeStruct((NC, NPAD, L), jnp.float32),
    mesh=_mesh,
    scratch_types=[
        pltpu.VMEM((CHUNK,), jnp.int32),
        pltpu.VMEM((CHUNK, L), jnp.float32),
        pltpu.VMEM((RPS, L), jnp.float32),
        pltpu.VMEM_SHARED((NPAD, L), jnp.float32),
    ],
)
def _deg_sc(dst_hbm, out_hbm, didx, ones_v, stage_v, acc_sh):
    cid = lax.axis_index("c")
    sid = lax.axis_index("s")
    wid = _my_wid()

    one = jnp.ones((L,), jnp.float32)

    def fill(i, _):
        ones_v[i, :] = one
        return 0

    lax.fori_loop(0, CHUNK, fill, 0)
    _zero_rows(stage_v)
    pltpu.sync_copy(stage_v, acc_sh.at[pl.ds(sid * RPS, RPS)])
    plsc.subcore_barrier()

    base = wid * EPT

    def chunk(j, _):
        pltpu.sync_copy(dst_hbm.at[pl.ds(base + j * CHUNK, CHUNK)], didx)
        pltpu.sync_copy(ones_v, acc_sh.at[didx], add=True)
        return 0

    lax.fori_loop(0, NCHUNK, chunk, 0)
    _acc_epilogue(acc_sh, stage_v, out_hbm, cid, sid)


@functools.partial(
    pl.kernel,
    out_type=jax.ShapeDtypeStruct((NC, NPAD, L), jnp.float32),
    mesh=_mesh,
    scratch_types=[
        pltpu.VMEM((CHUNK,), jnp.int32),
        pltpu.VMEM((CHUNK,), jnp.int32),
        pltpu.VMEM((CHUNK, L), jnp.float32),
        pltpu.VMEM((RPS, L), jnp.float32),
        pltpu.VMEM_SHARED((NPAD, L), jnp.float32),
        pltpu.SemaphoreType.DMA,
    ],
)
def _agg_sc(rows_hbm, src_hbm, dst_hbm, out_hbm,
            sidx, didx, rows_v, stage_v, acc_sh, sem):
    cid = lax.axis_index("c")
    sid = lax.axis_index("s")
    wid = _my_wid()

    _zero_rows(stage_v)
    pltpu.sync_copy(stage_v, acc_sh.at[pl.ds(sid * RPS, RPS)])
    plsc.subcore_barrier()

    base = wid * EPT

    def chunk(j, _):
        off = base + j * CHUNK
        pltpu.sync_copy(src_hbm.at[pl.ds(off, CHUNK)], sidx)
        pltpu.sync_copy(dst_hbm.at[pl.ds(off, CHUNK)], didx)
        pltpu.async_copy(rows_hbm.at[sidx], rows_v, sem).wait()
        pltpu.sync_copy(rows_v, acc_sh.at[didx], add=True)
        return 0

    lax.fori_loop(0, NCHUNK, chunk, 0)
    _acc_epilogue(acc_sh, stage_v, out_hbm, cid, sid)


_MB = 1024


def _mm1_body(x_ref, w_ref, o_ref):
    o_ref[...] = jnp.dot(x_ref[...], w_ref[...],
                         preferred_element_type=jnp.float32)


def _mm1(x_pad, w1):
    return pl.pallas_call(
        _mm1_body,
        grid=(NPAD // _MB,),
        in_specs=[
            pl.BlockSpec((_MB, 256), lambda i: (i, 0)),
            pl.BlockSpec((256, L), lambda i: (0, 0)),
        ],
        out_specs=pl.BlockSpec((_MB, L), lambda i: (i, 0)),
        out_shape=jax.ShapeDtypeStruct((NPAD, L), jnp.float32),
    )(x_pad, w1)


def _prep_body(dd_ref, hw_ref, g_ref, dis_ref):
    deg = dd_ref[0] + dd_ref[1] + 1.0
    dis = lax.rsqrt(deg)
    dis_ref[...] = dis
    g_ref[...] = hw_ref[...] * dis


def _prep(dd, hw1):
    return pl.pallas_call(
        _prep_body,
        grid=(NPAD // _MB,),
        in_specs=[
            pl.BlockSpec((NC, _MB, L), lambda i: (0, i, 0)),
            pl.BlockSpec((_MB, L), lambda i: (i, 0)),
        ],
        out_specs=[
            pl.BlockSpec((_MB, L), lambda i: (i, 0)),
            pl.BlockSpec((_MB, L), lambda i: (i, 0)),
        ],
        out_shape=[
            jax.ShapeDtypeStruct((NPAD, L), jnp.float32),
            jax.ShapeDtypeStruct((NPAD, L), jnp.float32),
        ],
    )(dd, hw1)


def _mid_body(a_ref, hw_ref, dis_ref, b1_ref, h_ref, g2_ref):
    dis = dis_ref[...]
    agg = a_ref[0] + a_ref[1]
    h = jnp.maximum(dis * agg + dis * dis * hw_ref[...] + b1_ref[...], 0.0)
    h_ref[...] = h
    g2_ref[...] = h * dis


def _mid(a1, hw1, dis, b1r):
    return pl.pallas_call(
        _mid_body,
        grid=(NPAD // _MB,),
        in_specs=[
            pl.BlockSpec((NC, _MB, L), lambda i: (0, i, 0)),
            pl.BlockSpec((_MB, L), lambda i: (i, 0)),
            pl.BlockSpec((_MB, L), lambda i: (i, 0)),
            pl.BlockSpec((1, L), lambda i: (0, 0)),
        ],
        out_specs=[
            pl.BlockSpec((_MB, L), lambda i: (i, 0)),
            pl.BlockSpec((_MB, L), lambda i: (i, 0)),
        ],
        out_shape=[
            jax.ShapeDtypeStruct((NPAD, L), jnp.float32),
            jax.ShapeDtypeStruct((NPAD, L), jnp.float32),
        ],
    )(a1, hw1, dis, b1r)


def _fin_body(a_ref, h_ref, dis_ref, w2_ref, b2_ref, o_ref):
    dis = dis_ref[...]
    m = dis * (a_ref[0] + a_ref[1]) + dis * dis * h_ref[...]
    z = jnp.dot(m, w2_ref[...], preferred_element_type=jnp.float32)
    z = z + b2_ref[...]
    zmax = jnp.max(z, axis=1, keepdims=True)
    ez = jnp.exp(z - zmax)
    lse = jnp.log(jnp.sum(ez, axis=1, keepdims=True)) + zmax
    o_ref[...] = z - lse


def _fin(a2, h1, dis, w2, b2r):
    return pl.pallas_call(
        _fin_body,
        grid=(NPAD // _MB,),
        in_specs=[
            pl.BlockSpec((NC, _MB, L), lambda i: (0, i, 0)),
            pl.BlockSpec((_MB, L), lambda i: (i, 0)),
            pl.BlockSpec((_MB, L), lambda i: (i, 0)),
            pl.BlockSpec((L, 256), lambda i: (0, 0)),
            pl.BlockSpec((1, 256), lambda i: (0, 0)),
        ],
        out_specs=pl.BlockSpec((_MB, 256), lambda i: (i, 0)),
        out_shape=jax.ShapeDtypeStruct((NPAD, 256), jnp.float32),
    )(a2, h1, dis, w2, b2r)


def kernel(x, edge_index, W1, b1, W2, b2):
    x_pad = jnp.zeros((NPAD, 256), jnp.float32).at[:N].set(x)
    src = jnp.zeros((EPAD,), jnp.int32).at[:E].set(edge_index[0])
    dst = jnp.full((EPAD,), N, jnp.int32).at[:E].set(edge_index[1])
    b1r = b1.reshape(1, L)
    b2r = b2.reshape(1, 256)

    dd = _deg_sc(dst)
    hw1 = _mm1(x_pad, W1)
    g1, dis = _prep(dd, hw1)
    a1 = _agg_sc(g1, src, dst)
    h1, g2 = _mid(a1, hw1, dis, b1r)
    a2 = _agg_sc(g2, src, dst)
    out = _fin(a2, h1, dis, W2, b2r)
    return out[:N]


# speedup vs baseline: 17.2050x; 17.2050x over previous
import functools

import jax
import jax.numpy as jnp
from jax import lax
from jax.experimental import pallas as pl
from jax.experimental.pallas import tpu as pltpu
from jax.experimental.pallas import tpu_sc as plsc

N = 10000
NPAD = 10240
E = 160000
NC, NS, L = 2, 16, 16
NW = NC * NS
CHUNK = 128
EPT = 5120
EPAD = EPT * NW
NCHUNK = EPT // CHUNK
RPS = NPAD // NS

_mesh = plsc.VectorSubcoreMesh(core_axis_name="c", subcore_axis_name="s")
_sc_params = pltpu.CompilerParams(use_tc_tiling_on_sc=False)


def _zero_rows(stage_v):
    z = jnp.zeros((L,), jnp.float32)

    def body(i, _):
        stage_v[i, :] = z
        return 0

    lax.fori_loop(0, RPS, body, 0)


def _my_wid():
    return lax.axis_index("s") * NC + lax.axis_index("c")


def _acc_epilogue(acc_sh, stage_v, out_hbm, cid, sid):
    plsc.subcore_barrier()
    pltpu.sync_copy(acc_sh.at[pl.ds(sid * RPS, RPS)], stage_v)
    pltpu.sync_copy(stage_v, out_hbm.at[cid].at[pl.ds(sid * RPS, RPS)])


@functools.partial(
    pl.kernel,
    out_type=jax.ShapeDtypeStruct((NC, NPAD, L), jnp.float32),
    mesh=_mesh,
    compiler_params=_sc_params,
    scratch_types=[
        pltpu.VMEM((CHUNK,), jnp.int32),
        pltpu.VMEM((CHUNK, L), jnp.float32),
        pltpu.VMEM((RPS, L), jnp.float32),
        pltpu.VMEM_SHARED((NPAD, L), jnp.float32),
    ],
)
def _deg_sc(dst_hbm, out_hbm, didx, ones_v, stage_v, acc_sh):
    cid = lax.axis_index("c")
    sid = lax.axis_index("s")
    wid = _my_wid()

    one = jnp.ones((L,), jnp.float32)

    def fill(i, _):
        ones_v[i, :] = one
        return 0

    lax.fori_loop(0, CHUNK, fill, 0)
    _zero_rows(stage_v)
    pltpu.sync_copy(stage_v, acc_sh.at[pl.ds(sid * RPS, RPS)])
    plsc.subcore_barrier()

    base = wid * EPT

    def chunk(j, _):
        pltpu.sync_copy(dst_hbm.at[pl.ds(base + j * CHUNK, CHUNK)], didx)
        pltpu.sync_copy(ones_v, acc_sh.at[didx], add=True)
        return 0

    lax.fori_loop(0, NCHUNK, chunk, 0)
    _acc_epilogue(acc_sh, stage_v, out_hbm, cid, sid)


@functools.partial(
    pl.kernel,
    out_type=jax.ShapeDtypeStruct((NC, NPAD, L), jnp.float32),
    mesh=_mesh,
    compiler_params=_sc_params,
    scratch_types=[
        pltpu.VMEM((CHUNK,), jnp.int32),
        pltpu.VMEM((CHUNK,), jnp.int32),
        pltpu.VMEM((CHUNK, L), jnp.float32),
        pltpu.VMEM((RPS, L), jnp.float32),
        pltpu.VMEM_SHARED((NPAD, L), jnp.float32),
        pltpu.SemaphoreType.DMA,
    ],
)
def _agg_sc(rows_hbm, src_hbm, dst_hbm, out_hbm,
            sidx, didx, rows_v, stage_v, acc_sh, sem):
    cid = lax.axis_index("c")
    sid = lax.axis_index("s")
    wid = _my_wid()

    _zero_rows(stage_v)
    pltpu.sync_copy(stage_v, acc_sh.at[pl.ds(sid * RPS, RPS)])
    plsc.subcore_barrier()

    base = wid * EPT

    def chunk(j, _):
        off = base + j * CHUNK
        pltpu.sync_copy(src_hbm.at[pl.ds(off, CHUNK)], sidx)
        pltpu.sync_copy(dst_hbm.at[pl.ds(off, CHUNK)], didx)
        pltpu.async_copy(rows_hbm.at[sidx], rows_v, sem).wait()
        pltpu.sync_copy(rows_v, acc_sh.at[didx], add=True)
        return 0

    lax.fori_loop(0, NCHUNK, chunk, 0)
    _acc_epilogue(acc_sh, stage_v, out_hbm, cid, sid)


_MB = 1024


def _mm1_body(x_ref, w_ref, o_ref):
    o_ref[...] = jnp.dot(x_ref[...], w_ref[...],
                         preferred_element_type=jnp.float32)


def _mm1(x_pad, w1):
    return pl.pallas_call(
        _mm1_body,
        grid=(NPAD // _MB,),
        in_specs=[
            pl.BlockSpec((_MB, 256), lambda i: (i, 0)),
            pl.BlockSpec((256, L), lambda i: (0, 0)),
        ],
        out_specs=pl.BlockSpec((_MB, L), lambda i: (i, 0)),
        out_shape=jax.ShapeDtypeStruct((NPAD, L), jnp.float32),
    )(x_pad, w1)


def _prep_body(dd_ref, hw_ref, g_ref, dis_ref):
    deg = dd_ref[0] + dd_ref[1] + 1.0
    dis = lax.rsqrt(deg)
    dis_ref[...] = dis
    g_ref[...] = hw_ref[...] * dis


def _prep(dd, hw1):
    return pl.pallas_call(
        _prep_body,
        grid=(NPAD // _MB,),
        in_specs=[
            pl.BlockSpec((NC, _MB, L), lambda i: (0, i, 0)),
            pl.BlockSpec((_MB, L), lambda i: (i, 0)),
        ],
        out_specs=[
            pl.BlockSpec((_MB, L), lambda i: (i, 0)),
            pl.BlockSpec((_MB, L), lambda i: (i, 0)),
        ],
        out_shape=[
            jax.ShapeDtypeStruct((NPAD, L), jnp.float32),
            jax.ShapeDtypeStruct((NPAD, L), jnp.float32),
        ],
    )(dd, hw1)


def _mid_body(a_ref, hw_ref, dis_ref, b1_ref, h_ref, g2_ref):
    dis = dis_ref[...]
    agg = a_ref[0] + a_ref[1]
    h = jnp.maximum(dis * agg + dis * dis * hw_ref[...] + b1_ref[...], 0.0)
    h_ref[...] = h
    g2_ref[...] = h * dis


def _mid(a1, hw1, dis, b1r):
    return pl.pallas_call(
        _mid_body,
        grid=(NPAD // _MB,),
        in_specs=[
            pl.BlockSpec((NC, _MB, L), lambda i: (0, i, 0)),
            pl.BlockSpec((_MB, L), lambda i: (i, 0)),
            pl.BlockSpec((_MB, L), lambda i: (i, 0)),
            pl.BlockSpec((1, L), lambda i: (0, 0)),
        ],
        out_specs=[
            pl.BlockSpec((_MB, L), lambda i: (i, 0)),
            pl.BlockSpec((_MB, L), lambda i: (i, 0)),
        ],
        out_shape=[
            jax.ShapeDtypeStruct((NPAD, L), jnp.float32),
            jax.ShapeDtypeStruct((NPAD, L), jnp.float32),
        ],
    )(a1, hw1, dis, b1r)


def _fin_body(a_ref, h_ref, dis_ref, w2_ref, b2_ref, o_ref):
    dis = dis_ref[...]
    m = dis * (a_ref[0] + a_ref[1]) + dis * dis * h_ref[...]
    z = jnp.dot(m, w2_ref[...], preferred_element_type=jnp.float32)
    z = z + b2_ref[...]
    zmax = jnp.max(z, axis=1, keepdims=True)
    ez = jnp.exp(z - zmax)
    lse = jnp.log(jnp.sum(ez, axis=1, keepdims=True)) + zmax
    o_ref[...] = z - lse


def _fin(a2, h1, dis, w2, b2r):
    return pl.pallas_call(
        _fin_body,
        grid=(NPAD // _MB,),
        in_specs=[
            pl.BlockSpec((NC, _MB, L), lambda i: (0, i, 0)),
            pl.BlockSpec((_MB, L), lambda i: (i, 0)),
            pl.BlockSpec((_MB, L), lambda i: (i, 0)),
            pl.BlockSpec((L, 256), lambda i: (0, 0)),
            pl.BlockSpec((1, 256), lambda i: (0, 0)),
        ],
        out_specs=pl.BlockSpec((_MB, 256), lambda i: (i, 0)),
        out_shape=jax.ShapeDtypeStruct((NPAD, 256), jnp.float32),
    )(a2, h1, dis, w2, b2r)


def kernel(x, edge_index, W1, b1, W2, b2):
    x_pad = jnp.zeros((NPAD, 256), jnp.float32).at[:N].set(x)
    src = jnp.zeros((EPAD,), jnp.int32).at[:E].set(edge_index[0])
    dst = jnp.full((EPAD,), N, jnp.int32).at[:E].set(edge_index[1])
    b1r = b1.reshape(1, L)
    b2r = b2.reshape(1, 256)

    dd = _deg_sc(dst)
    hw1 = _mm1(x_pad, W1)
    g1, dis = _prep(dd, hw1)
    a1 = _agg_sc(g1, src, dst)
    h1, g2 = _mid(a1, hw1, dis, b1r)
    a2 = _agg_sc(g2, src, dst)
    out = _fin(a2, h1, dis, W2, b2r)
    return out[:N]


# speedup vs baseline: 28.9306x; 1.6815x over previous
import functools

import jax
import jax.numpy as jnp
from jax import lax
from jax.experimental import pallas as pl
from jax.experimental.pallas import tpu as pltpu
from jax.experimental.pallas import tpu_sc as plsc

N = 10000
NPAD = 10240
E = 160000
NC, NS, L = 2, 16, 16
NW = NC * NS
CHUNK = 128
EPT = 5120
EPAD = EPT * NW
NCHUNK = EPT // CHUNK
CPS = NCHUNK
RPS = NPAD // NS
LOOKAHEAD = 6

_mesh = plsc.VectorSubcoreMesh(core_axis_name="c", subcore_axis_name="s")
_sc_params = pltpu.CompilerParams(use_tc_tiling_on_sc=False)


def _zero_rows(stage_v):
    z = jnp.zeros((L,), jnp.float32)

    def body(i, _):
        stage_v[i, :] = z
        return 0

    lax.fori_loop(0, RPS, body, 0)


def _acc_epilogue(acc_sh, stage_v, out_hbm, cid, sid):
    plsc.subcore_barrier()
    pltpu.sync_copy(acc_sh.at[pl.ds(sid * RPS, RPS)], stage_v)
    pltpu.sync_copy(stage_v, out_hbm.at[cid].at[pl.ds(sid * RPS, RPS)])


@functools.partial(
    pl.kernel,
    out_type=jax.ShapeDtypeStruct((NC, NPAD, L), jnp.float32),
    mesh=_mesh,
    compiler_params=_sc_params,
    scratch_types=[
        pltpu.VMEM((CPS, CHUNK), jnp.int32),
        pltpu.VMEM((CHUNK, L), jnp.float32),
        pltpu.VMEM((RPS, L), jnp.float32),
        pltpu.VMEM_SHARED((NPAD, L), jnp.float32),
        pltpu.SemaphoreType.DMA,
    ],
)
def _deg_sc(dst_hbm, out_hbm, didx, ones_v, stage_v, acc_sh, sem_s):
    cid = lax.axis_index("c")
    sid = lax.axis_index("s")
    wid = sid * NC + cid

    one = jnp.ones((L,), jnp.float32)

    def fill(i, _):
        ones_v[i, :] = one
        return 0

    lax.fori_loop(0, CHUNK, fill, 0)
    _zero_rows(stage_v)
    pltpu.sync_copy(dst_hbm.at[pl.ds(wid * CPS, CPS)], didx)
    pltpu.sync_copy(stage_v, acc_sh.at[pl.ds(sid * RPS, RPS)])
    plsc.subcore_barrier()

    def fire(j, _):
        pltpu.async_copy(ones_v, acc_sh.at[didx.at[j]], sem_s, add=True)
        return 0

    lax.fori_loop(0, NCHUNK, fire, 0)

    def drain(j, _):
        pltpu.make_async_copy(ones_v, acc_sh.at[didx.at[j]], sem_s).wait()
        return 0

    lax.fori_loop(0, NCHUNK, drain, 0)
    _acc_epilogue(acc_sh, stage_v, out_hbm, cid, sid)


@functools.partial(
    pl.kernel,
    out_type=jax.ShapeDtypeStruct((NC, NPAD, L), jnp.float32),
    mesh=_mesh,
    compiler_params=_sc_params,
    scratch_types=[
        pltpu.VMEM((CPS, CHUNK), jnp.int32),
        pltpu.VMEM((CPS, CHUNK), jnp.int32),
        pltpu.VMEM((EPT, L), jnp.float32),
        pltpu.VMEM((RPS, L), jnp.float32),
        pltpu.VMEM_SHARED((NPAD, L), jnp.float32),
        pltpu.SemaphoreType.DMA,
        pltpu.SemaphoreType.DMA,
    ],
)
def _agg_sc(rows_hbm, src_hbm, dst_hbm, out_hbm,
            sidx, didx, rows_v, stage_v, acc_sh, sem_g, sem_s):
    cid = lax.axis_index("c")
    sid = lax.axis_index("s")
    wid = sid * NC + cid

    _zero_rows(stage_v)
    pltpu.sync_copy(src_hbm.at[pl.ds(wid * CPS, CPS)], sidx)
    pltpu.sync_copy(dst_hbm.at[pl.ds(wid * CPS, CPS)], didx)
    pltpu.sync_copy(stage_v, acc_sh.at[pl.ds(sid * RPS, RPS)])
    plsc.subcore_barrier()

    def gather(j):
        return pltpu.async_copy(
            rows_hbm.at[sidx.at[j]],
            rows_v.at[pl.ds(j * CHUNK, CHUNK)], sem_g)

    def scatter(j):
        return pltpu.async_copy(
            rows_v.at[pl.ds(j * CHUNK, CHUNK)],
            acc_sh.at[didx.at[j]], sem_s, add=True)

    def prime(j, _):
        gather(j)
        return 0

    lax.fori_loop(0, LOOKAHEAD, prime, 0)

    def step(j, _):
        @pl.when(j + LOOKAHEAD < NCHUNK)
        def _():
            gather(j + LOOKAHEAD)

        pltpu.make_async_copy(
            rows_hbm.at[sidx.at[j]],
            rows_v.at[pl.ds(j * CHUNK, CHUNK)], sem_g).wait()
        scatter(j)
        return 0

    lax.fori_loop(0, NCHUNK, step, 0)

    def drain(j, _):
        pltpu.make_async_copy(
            rows_v.at[pl.ds(j * CHUNK, CHUNK)],
            acc_sh.at[didx.at[j]], sem_s).wait()
        return 0

    lax.fori_loop(0, NCHUNK, drain, 0)
    _acc_epilogue(acc_sh, stage_v, out_hbm, cid, sid)


_MB = 1024


def _mm1_body(x_ref, w_ref, o_ref):
    o_ref[...] = jnp.dot(x_ref[...], w_ref[...],
                         preferred_element_type=jnp.float32)


def _mm1(x, w1):
    return pl.pallas_call(
        _mm1_body,
        grid=(NPAD // _MB,),
        in_specs=[
            pl.BlockSpec((_MB, 256), lambda i: (i, 0)),
            pl.BlockSpec((256, L), lambda i: (0, 0)),
        ],
        out_specs=pl.BlockSpec((_MB, L), lambda i: (i, 0)),
        out_shape=jax.ShapeDtypeStruct((NPAD, L), jnp.float32),
    )(x, w1)


def _prep_body(dd_ref, hw_ref, g_ref, dis_ref):
    deg = dd_ref[0] + dd_ref[1] + 1.0
    dis = lax.rsqrt(deg)
    dis_ref[...] = dis
    g_ref[...] = hw_ref[...] * dis


def _prep(dd, hw1):
    return pl.pallas_call(
        _prep_body,
        grid=(NPAD // _MB,),
        in_specs=[
            pl.BlockSpec((NC, _MB, L), lambda i: (0, i, 0)),
            pl.BlockSpec((_MB, L), lambda i: (i, 0)),
        ],
        out_specs=[
            pl.BlockSpec((_MB, L), lambda i: (i, 0)),
            pl.BlockSpec((_MB, L), lambda i: (i, 0)),
        ],
        out_shape=[
            jax.ShapeDtypeStruct((NPAD, L), jnp.float32),
            jax.ShapeDtypeStruct((NPAD, L), jnp.float32),
        ],
    )(dd, hw1)


def _mid_body(a_ref, hw_ref, dis_ref, b1_ref, h_ref, g2_ref):
    dis = dis_ref[...]
    agg = a_ref[0] + a_ref[1]
    h = jnp.maximum(dis * agg + dis * dis * hw_ref[...] + b1_ref[...], 0.0)
    h_ref[...] = h
    g2_ref[...] = h * dis


def _mid(a1, hw1, dis, b1r):
    return pl.pallas_call(
        _mid_body,
        grid=(NPAD // _MB,),
        in_specs=[
            pl.BlockSpec((NC, _MB, L), lambda i: (0, i, 0)),
            pl.BlockSpec((_MB, L), lambda i: (i, 0)),
            pl.BlockSpec((_MB, L), lambda i: (i, 0)),
            pl.BlockSpec((1, L), lambda i: (0, 0)),
        ],
        out_specs=[
            pl.BlockSpec((_MB, L), lambda i: (i, 0)),
            pl.BlockSpec((_MB, L), lambda i: (i, 0)),
        ],
        out_shape=[
            jax.ShapeDtypeStruct((NPAD, L), jnp.float32),
            jax.ShapeDtypeStruct((NPAD, L), jnp.float32),
        ],
    )(a1, hw1, dis, b1r)


def _fin_body(a_ref, h_ref, dis_ref, w2_ref, b2_ref, o_ref):
    dis = dis_ref[...]
    m = dis * (a_ref[0] + a_ref[1]) + dis * dis * h_ref[...]
    z = jnp.dot(m, w2_ref[...], preferred_element_type=jnp.float32)
    z = z + b2_ref[...]
    zmax = jnp.max(z, axis=1, keepdims=True)
    ez = jnp.exp(z - zmax)
    lse = jnp.log(jnp.sum(ez, axis=1, keepdims=True)) + zmax
    o_ref[...] = z - lse


def _fin(a2, h1, dis, w2, b2r):
    return pl.pallas_call(
        _fin_body,
        grid=(NPAD // _MB,),
        in_specs=[
            pl.BlockSpec((NC, _MB, L), lambda i: (0, i, 0)),
            pl.BlockSpec((_MB, L), lambda i: (i, 0)),
            pl.BlockSpec((_MB, L), lambda i: (i, 0)),
            pl.BlockSpec((L, 256), lambda i: (0, 0)),
            pl.BlockSpec((1, 256), lambda i: (0, 0)),
        ],
        out_specs=pl.BlockSpec((_MB, 256), lambda i: (i, 0)),
        out_shape=jax.ShapeDtypeStruct((N, 256), jnp.float32),
    )(a2, h1, dis, w2, b2r)


def kernel(x, edge_index, W1, b1, W2, b2):
    x_pad = jnp.zeros((NPAD, 256), jnp.float32).at[:N].set(x)
    src = (jnp.zeros((EPAD,), jnp.int32).at[:E].set(edge_index[0])
           .reshape(EPAD // CHUNK, CHUNK))
    dst = (jnp.full((EPAD,), N, jnp.int32).at[:E].set(edge_index[1])
           .reshape(EPAD // CHUNK, CHUNK))
    b1r = b1.reshape(1, L)
    b2r = b2.reshape(1, 256)

    dd = _deg_sc(dst)
    hw1 = _mm1(x_pad, W1)
    g1, dis = _prep(dd, hw1)
    a1 = _agg_sc(g1, src, dst)
    h1, g2 = _mid(a1, hw1, dis, b1r)
    a2 = _agg_sc(g2, src, dst)
    return _fin(a2, h1, dis, W2, b2r)


# speedup vs baseline: 35.8555x; 1.2394x over previous
import functools

import jax
import jax.numpy as jnp
from jax import lax
from jax.experimental import pallas as pl
from jax.experimental.pallas import tpu as pltpu
from jax.experimental.pallas import tpu_sc as plsc

N = 10000
NPAD = 10240
E = 160000
NC, NS, L = 2, 16, 16
NW = NC * NS
CHUNK = 128
EPT = 5120
EPAD = EPT * NW
NCHUNK = EPT // CHUNK
CPS = NCHUNK
RPS = NPAD // NS
LOOKAHEAD = 6

_mesh = plsc.VectorSubcoreMesh(core_axis_name="c", subcore_axis_name="s")
_sc_params = pltpu.CompilerParams(use_tc_tiling_on_sc=False)


def _zero_rows(stage_v):
    z = jnp.zeros((L,), jnp.float32)

    def body(i, _):
        stage_v[i, :] = z
        return 0

    lax.fori_loop(0, RPS, body, 0)


def _acc_epilogue(acc_sh, stage_v, out_hbm, cid, sid):
    plsc.subcore_barrier()
    pltpu.sync_copy(acc_sh.at[pl.ds(sid * RPS, RPS)], stage_v)
    pltpu.sync_copy(stage_v, out_hbm.at[cid].at[pl.ds(sid * RPS, RPS)])


@functools.partial(
    pl.kernel,
    out_type=jax.ShapeDtypeStruct((NC, NPAD, L), jnp.float32),
    mesh=_mesh,
    compiler_params=_sc_params,
    scratch_types=[
        pltpu.VMEM((CPS, CHUNK), jnp.int32),
        pltpu.VMEM((CHUNK, L), jnp.float32),
        pltpu.VMEM((RPS, L), jnp.float32),
        pltpu.VMEM_SHARED((NPAD, L), jnp.float32),
        pltpu.SemaphoreType.DMA,
    ],
)
def _deg_sc(dst_hbm, out_hbm, didx, ones_v, stage_v, acc_sh, sem_s):
    cid = lax.axis_index("c")
    sid = lax.axis_index("s")
    wid = sid * NC + cid

    one = jnp.ones((L,), jnp.float32)

    def fill(i, _):
        ones_v[i, :] = one
        return 0

    lax.fori_loop(0, CHUNK, fill, 0)
    _zero_rows(stage_v)
    pltpu.sync_copy(dst_hbm.at[pl.ds(wid * CPS, CPS)], didx)
    pltpu.sync_copy(stage_v, acc_sh.at[pl.ds(sid * RPS, RPS)])
    plsc.subcore_barrier()

    def fire(j, _):
        pltpu.async_copy(ones_v, acc_sh.at[didx.at[j]], sem_s, add=True)
        return 0

    lax.fori_loop(0, NCHUNK, fire, 0)

    def drain(j, _):
        pltpu.make_async_copy(ones_v, acc_sh.at[didx.at[j]], sem_s).wait()
        return 0

    lax.fori_loop(0, NCHUNK, drain, 0)
    _acc_epilogue(acc_sh, stage_v, out_hbm, cid, sid)


@functools.partial(
    pl.kernel,
    out_type=jax.ShapeDtypeStruct((NC, NPAD, L), jnp.float32),
    mesh=_mesh,
    compiler_params=_sc_params,
    scratch_types=[
        pltpu.VMEM((CPS, CHUNK), jnp.int32),
        pltpu.VMEM((CPS, CHUNK), jnp.int32),
        pltpu.VMEM((EPT, L), jnp.float32),
        pltpu.VMEM((RPS, L), jnp.float32),
        pltpu.VMEM_SHARED((NPAD, L), jnp.float32),
        pltpu.VMEM_SHARED((NPAD, L), jnp.float32),
        pltpu.SemaphoreType.DMA,
        pltpu.SemaphoreType.DMA,
    ],
)
def _agg_sc(rows_hbm, src_hbm, dst_hbm, out_hbm,
            sidx, didx, rows_v, stage_v, tab_sh, acc_sh, sem_g, sem_s):
    cid = lax.axis_index("c")
    sid = lax.axis_index("s")
    wid = sid * NC + cid

    pltpu.sync_copy(rows_hbm.at[pl.ds(sid * RPS, RPS)], stage_v)
    pltpu.sync_copy(stage_v, tab_sh.at[pl.ds(sid * RPS, RPS)])
    _zero_rows(stage_v)
    pltpu.sync_copy(src_hbm.at[pl.ds(wid * CPS, CPS)], sidx)
    pltpu.sync_copy(dst_hbm.at[pl.ds(wid * CPS, CPS)], didx)
    pltpu.sync_copy(stage_v, acc_sh.at[pl.ds(sid * RPS, RPS)])
    plsc.subcore_barrier()

    def gather(j):
        return pltpu.async_copy(
            tab_sh.at[sidx.at[j]],
            rows_v.at[pl.ds(j * CHUNK, CHUNK)], sem_g)

    def scatter(j):
        return pltpu.async_copy(
            rows_v.at[pl.ds(j * CHUNK, CHUNK)],
            acc_sh.at[didx.at[j]], sem_s, add=True)

    def prime(j, _):
        gather(j)
        return 0

    lax.fori_loop(0, LOOKAHEAD, prime, 0)

    def step(j, _):
        @pl.when(j + LOOKAHEAD < NCHUNK)
        def _():
            gather(j + LOOKAHEAD)

        pltpu.make_async_copy(
            tab_sh.at[sidx.at[j]],
            rows_v.at[pl.ds(j * CHUNK, CHUNK)], sem_g).wait()
        scatter(j)
        return 0

    lax.fori_loop(0, NCHUNK, step, 0)

    def drain(j, _):
        pltpu.make_async_copy(
            rows_v.at[pl.ds(j * CHUNK, CHUNK)],
            acc_sh.at[didx.at[j]], sem_s).wait()
        return 0

    lax.fori_loop(0, NCHUNK, drain, 0)
    _acc_epilogue(acc_sh, stage_v, out_hbm, cid, sid)


_MB = 1024


def _mm1_body(x_ref, w_ref, o_ref):
    o_ref[...] = jnp.dot(x_ref[...], w_ref[...],
                         preferred_element_type=jnp.float32)


def _mm1(x, w1):
    return pl.pallas_call(
        _mm1_body,
        grid=(NPAD // _MB,),
        in_specs=[
            pl.BlockSpec((_MB, 256), lambda i: (i, 0)),
            pl.BlockSpec((256, L), lambda i: (0, 0)),
        ],
        out_specs=pl.BlockSpec((_MB, L), lambda i: (i, 0)),
        out_shape=jax.ShapeDtypeStruct((NPAD, L), jnp.float32),
    )(x, w1)


def _prep_body(dd_ref, hw_ref, g_ref, dis_ref):
    deg = dd_ref[0] + dd_ref[1] + 1.0
    dis = lax.rsqrt(deg)
    dis_ref[...] = dis
    g_ref[...] = hw_ref[...] * dis


def _prep(dd, hw1):
    return pl.pallas_call(
        _prep_body,
        grid=(NPAD // _MB,),
        in_specs=[
            pl.BlockSpec((NC, _MB, L), lambda i: (0, i, 0)),
            pl.BlockSpec((_MB, L), lambda i: (i, 0)),
        ],
        out_specs=[
            pl.BlockSpec((_MB, L), lambda i: (i, 0)),
            pl.BlockSpec((_MB, L), lambda i: (i, 0)),
        ],
        out_shape=[
            jax.ShapeDtypeStruct((NPAD, L), jnp.float32),
            jax.ShapeDtypeStruct((NPAD, L), jnp.float32),
        ],
    )(dd, hw1)


def _mid_body(a_ref, hw_ref, dis_ref, b1_ref, h_ref, g2_ref):
    dis = dis_ref[...]
    agg = a_ref[0] + a_ref[1]
    h = jnp.maximum(dis * agg + dis * dis * hw_ref[...] + b1_ref[...], 0.0)
    h_ref[...] = h
    g2_ref[...] = h * dis


def _mid(a1, hw1, dis, b1r):
    return pl.pallas_call(
        _mid_body,
        grid=(NPAD // _MB,),
        in_specs=[
            pl.BlockSpec((NC, _MB, L), lambda i: (0, i, 0)),
            pl.BlockSpec((_MB, L), lambda i: (i, 0)),
            pl.BlockSpec((_MB, L), lambda i: (i, 0)),
            pl.BlockSpec((1, L), lambda i: (0, 0)),
        ],
        out_specs=[
            pl.BlockSpec((_MB, L), lambda i: (i, 0)),
            pl.BlockSpec((_MB, L), lambda i: (i, 0)),
        ],
        out_shape=[
            jax.ShapeDtypeStruct((NPAD, L), jnp.float32),
            jax.ShapeDtypeStruct((NPAD, L), jnp.float32),
        ],
    )(a1, hw1, dis, b1r)


def _fin_body(a_ref, h_ref, dis_ref, w2_ref, b2_ref, o_ref):
    dis = dis_ref[...]
    m = dis * (a_ref[0] + a_ref[1]) + dis * dis * h_ref[...]
    z = jnp.dot(m, w2_ref[...], preferred_element_type=jnp.float32)
    z = z + b2_ref[...]
    zmax = jnp.max(z, axis=1, keepdims=True)
    ez = jnp.exp(z - zmax)
    lse = jnp.log(jnp.sum(ez, axis=1, keepdims=True)) + zmax
    o_ref[...] = z - lse


def _fin(a2, h1, dis, w2, b2r):
    return pl.pallas_call(
        _fin_body,
        grid=(NPAD // _MB,),
        in_specs=[
            pl.BlockSpec((NC, _MB, L), lambda i: (0, i, 0)),
            pl.BlockSpec((_MB, L), lambda i: (i, 0)),
            pl.BlockSpec((_MB, L), lambda i: (i, 0)),
            pl.BlockSpec((L, 256), lambda i: (0, 0)),
            pl.BlockSpec((1, 256), lambda i: (0, 0)),
        ],
        out_specs=pl.BlockSpec((_MB, 256), lambda i: (i, 0)),
        out_shape=jax.ShapeDtypeStruct((N, 256), jnp.float32),
    )(a2, h1, dis, w2, b2r)


def kernel(x, edge_index, W1, b1, W2, b2):
    src = (jnp.zeros((EPAD,), jnp.int32).at[:E].set(edge_index[0])
           .reshape(EPAD // CHUNK, CHUNK))
    dst = (jnp.full((EPAD,), N, jnp.int32).at[:E].set(edge_index[1])
           .reshape(EPAD // CHUNK, CHUNK))
    b1r = b1.reshape(1, L)
    b2r = b2.reshape(1, 256)

    dd = _deg_sc(dst)
    hw1 = _mm1(x, W1)
    g1, dis = _prep(dd, hw1)
    a1 = _agg_sc(g1, src, dst)
    h1, g2 = _mid(a1, hw1, dis, b1r)
    a2 = _agg_sc(g2, src, dst)
    return _fin(a2, h1, dis, W2, b2r)


# speedup vs baseline: 40.7669x; 1.1370x over previous
import functools

import jax
import jax.numpy as jnp
from jax import lax
from jax.experimental import pallas as pl
from jax.experimental.pallas import tpu as pltpu
from jax.experimental.pallas import tpu_sc as plsc

N = 10000
NPAD = 10240
E = 160000
NC, NS, L = 2, 16, 16
NW = NC * NS
CHUNK = 128
EPT = 5120
EPAD = EPT * NW
NCHUNK = EPT // CHUNK
CPS = NCHUNK
ROWS = EPAD // CHUNK
EPSC = ROWS // NS
RPS = NPAD // NS
RING = 16
LOOK = 6

_mesh = plsc.VectorSubcoreMesh(core_axis_name="c", subcore_axis_name="s")
_sc_params = pltpu.CompilerParams(use_tc_tiling_on_sc=False,
                                  needs_layout_passes=False)


def _fill_rows(buf, nrows, vec):
    def body(i, _):
        buf[i, :] = vec
        return 0

    lax.fori_loop(0, nrows, body, 0)


def _rsqrt16(x):
    i = plsc.bitcast(x, jnp.int32)
    i = jnp.int32(0x5F3759DF) - (i >> 1)
    y = plsc.bitcast(i, jnp.float32)
    xh = x * 0.5
    y = y * (1.5 - xh * y * y)
    y = y * (1.5 - xh * y * y)
    y = y * (1.5 - xh * y * y)
    return y


def _agg_pipeline(sidx, didx, rows_v, tab_sh, acc_sh, sem_g, sem_s):
    def rows_at(j):
        return rows_v.at[pl.ds(lax.rem(j, RING) * CHUNK, CHUNK)]

    def gather(j):
        pltpu.async_copy(tab_sh.at[sidx.at[j]], rows_at(j), sem_g)

    def wait_gather(j):
        pltpu.make_async_copy(tab_sh.at[sidx.at[j]], rows_at(j), sem_g).wait()

    def scatter(j):
        pltpu.async_copy(rows_at(j), acc_sh.at[didx.at[j]], sem_s, add=True)

    def wait_scatter(j):
        pltpu.make_async_copy(rows_at(j), acc_sh.at[didx.at[j]], sem_s).wait()

    def prime(j, _):
        gather(j)
        return 0

    lax.fori_loop(0, LOOK, prime, 0)

    def step(j, _):
        @pl.when(j >= RING - LOOK)
        def _():
            wait_scatter(j - (RING - LOOK))

        @pl.when(j + LOOK < NCHUNK)
        def _():
            gather(j + LOOK)

        wait_gather(j)
        scatter(j)
        return 0

    lax.fori_loop(0, NCHUNK, step, 0)

    def drain(j, _):
        wait_scatter(j)
        return 0

    lax.fori_loop(NCHUNK - (RING - LOOK), NCHUNK, drain, 0)


@functools.partial(
    pl.kernel,
    out_type=(
        jax.ShapeDtypeStruct((NC, NPAD, L), jnp.float32),
        jax.ShapeDtypeStruct((NPAD, L), jnp.float32),
    ),
    mesh=_mesh,
    compiler_params=_sc_params,
    scratch_types=[
        pltpu.VMEM((EPSC, CHUNK), jnp.int32),
        pltpu.VMEM((NPAD,), jnp.float32),
        pltpu.VMEM((NS, RPS), jnp.float32),
        pltpu.VMEM((RPS,), jnp.float32),
        pltpu.VMEM((RPS, L), jnp.float32),
        pltpu.VMEM((RPS, L), jnp.float32),
        pltpu.VMEM((CPS, CHUNK), jnp.int32),
        pltpu.VMEM((CPS, CHUNK), jnp.int32),
        pltpu.VMEM((RING * CHUNK, L), jnp.float32),
        pltpu.VMEM_SHARED((NPAD, L), jnp.float32),
        pltpu.VMEM_SHARED((NPAD, L), jnp.float32),
        pltpu.VMEM_SHARED((NS, NPAD), jnp.float32),
        pltpu.SemaphoreType.DMA,
        pltpu.SemaphoreType.DMA,
    ],
)
def _mega1(hw_hbm, src_hbm, dst_hbm, a_out, dis_out,
           didx_deg, hist_v, hbuf, disv, rowbuf, disrow,
           sidx, didx, rows_v, tab_sh, acc_sh, hist_sh, sem_g, sem_s):
    cid = lax.axis_index("c")
    sid = lax.axis_index("s")
    wid = sid * NC + cid
    zero16 = jnp.zeros((L,), jnp.float32)
    one16 = jnp.ones((L,), jnp.float32)

    pltpu.sync_copy(dst_hbm.at[pl.ds(sid * EPSC, EPSC)], didx_deg)
    pltpu.sync_copy(src_hbm.at[pl.ds(wid * CPS, CPS)], sidx)
    pltpu.sync_copy(dst_hbm.at[pl.ds(wid * CPS, CPS)], didx)
    pltpu.sync_copy(hw_hbm.at[pl.ds(sid * RPS, RPS)], rowbuf)

    def hzero(i, _):
        hist_v[pl.ds(i * L, L)] = zero16
        return 0

    lax.fori_loop(0, NPAD // L, hzero, 0)

    def hrow(r, _):
        def hvec(k, _):
            iv = didx_deg[r, pl.ds(k * L, L)]
            plsc.addupdate_scatter(hist_v, [iv], one16)
            return 0

        lax.fori_loop(0, CHUNK // L, hvec, 0)
        return 0

    lax.fori_loop(0, EPSC, hrow, 0)
    pltpu.sync_copy(hist_v, hist_sh.at[sid])
    plsc.subcore_barrier()

    def hload(p, _):
        pltpu.sync_copy(hist_sh.at[p].at[pl.ds(sid * RPS, RPS)], hbuf.at[p])
        return 0

    lax.fori_loop(0, NS, hload, 0)

    def dvec(c, _):
        def hsum(p, acc):
            return acc + hbuf[p, pl.ds(c * L, L)]

        deg = lax.fori_loop(0, NS, hsum, one16)
        disv[pl.ds(c * L, L)] = _rsqrt16(deg)
        return 0

    lax.fori_loop(0, RPS // L, dvec, 0)

    def grow(c, _):
        dv = disv[pl.ds(c * L, L)]
        for k in range(L):
            i = c * L + k
            srow = jnp.full((L,), dv[k], jnp.float32)
            disrow[i, :] = srow
            rowbuf[i, :] = rowbuf[i, :] * srow
        return 0

    lax.fori_loop(0, RPS // L, grow, 0)

    @pl.when(cid == 0)
    def _():
        pltpu.sync_copy(disrow, dis_out.at[pl.ds(sid * RPS, RPS)])

    pltpu.sync_copy(rowbuf, tab_sh.at[pl.ds(sid * RPS, RPS)])
    _fill_rows(disrow, RPS, zero16)
    pltpu.sync_copy(disrow, acc_sh.at[pl.ds(sid * RPS, RPS)])
    plsc.subcore_barrier()

    _agg_pipeline(sidx, didx, rows_v, tab_sh, acc_sh, sem_g, sem_s)

    plsc.subcore_barrier()
    pltpu.sync_copy(acc_sh.at[pl.ds(sid * RPS, RPS)], rowbuf)
    pltpu.sync_copy(rowbuf, a_out.at[cid].at[pl.ds(sid * RPS, RPS)])


@functools.partial(
    pl.kernel,
    out_type=(
        jax.ShapeDtypeStruct((NC, NPAD, L), jnp.float32),
        jax.ShapeDtypeStruct((NPAD, L), jnp.float32),
    ),
    mesh=_mesh,
    compiler_params=_sc_params,
    scratch_types=[
        pltpu.VMEM((RPS, L), jnp.float32),
        pltpu.VMEM((RPS, L), jnp.float32),
        pltpu.VMEM((RPS, L), jnp.float32),
        pltpu.VMEM((RPS, L), jnp.float32),
        pltpu.VMEM((L,), jnp.float32),
        pltpu.VMEM((CPS, CHUNK), jnp.int32),
        pltpu.VMEM((CPS, CHUNK), jnp.int32),
        pltpu.VMEM((RING * CHUNK, L), jnp.float32),
        pltpu.VMEM_SHARED((NPAD, L), jnp.float32),
        pltpu.VMEM_SHARED((NPAD, L), jnp.float32),
        pltpu.SemaphoreType.DMA,
        pltpu.SemaphoreType.DMA,
    ],
)
def _mega2(a1_hbm, hw_hbm, dis_hbm, b1_hbm, src_hbm, dst_hbm,
           a_out, h1_out,
           abuf0, abuf1, rowbuf, disrow, b1v,
           sidx, didx, rows_v, tab_sh, acc_sh, sem_g, sem_s):
    cid = lax.axis_index("c")
    sid = lax.axis_index("s")
    wid = sid * NC + cid
    zero16 = jnp.zeros((L,), jnp.float32)
    sl = pl.ds(sid * RPS, RPS)

    pltpu.sync_copy(a1_hbm.at[0].at[sl], abuf0)
    pltpu.sync_copy(a1_hbm.at[1].at[sl], abuf1)
    pltpu.sync_copy(hw_hbm.at[sl], rowbuf)
    pltpu.sync_copy(dis_hbm.at[sl], disrow)
    pltpu.sync_copy(b1_hbm, b1v)
    pltpu.sync_copy(src_hbm.at[pl.ds(wid * CPS, CPS)], sidx)
    pltpu.sync_copy(dst_hbm.at[pl.ds(wid * CPS, CPS)], didx)
    b1 = b1v[...]

    def hrow(i, _):
        d = disrow[i, :]
        agg = abuf0[i, :] + abuf1[i, :]
        h = jnp.maximum(d * agg + d * d * rowbuf[i, :] + b1, 0.0)
        abuf0[i, :] = h
        rowbuf[i, :] = h * d
        return 0

    lax.fori_loop(0, RPS, hrow, 0)

    @pl.when(cid == 0)
    def _():
        pltpu.sync_copy(abuf0, h1_out.at[sl])

    pltpu.sync_copy(rowbuf, tab_sh.at[sl])
    _fill_rows(abuf1, RPS, zero16)
    pltpu.sync_copy(abuf1, acc_sh.at[sl])
    plsc.subcore_barrier()

    _agg_pipeline(sidx, didx, rows_v, tab_sh, acc_sh, sem_g, sem_s)

    plsc.subcore_barrier()
    pltpu.sync_copy(acc_sh.at[sl], rowbuf)
    pltpu.sync_copy(rowbuf, a_out.at[cid].at[sl])


_MB = 1024


def _mm1_body(x_ref, w_ref, o_ref):
    o_ref[...] = jnp.dot(x_ref[...], w_ref[...],
                         preferred_element_type=jnp.float32)


def _mm1(x, w1):
    return pl.pallas_call(
        _mm1_body,
        grid=(NPAD // _MB,),
        in_specs=[
            pl.BlockSpec((_MB, 256), lambda i: (i, 0)),
            pl.BlockSpec((256, L), lambda i: (0, 0)),
        ],
        out_specs=pl.BlockSpec((_MB, L), lambda i: (i, 0)),
        out_shape=jax.ShapeDtypeStruct((NPAD, L), jnp.float32),
    )(x, w1)


def _fin_body(a_ref, h_ref, dis_ref, w2_ref, b2_ref, o_ref):
    dis = dis_ref[...]
    m = dis * (a_ref[0] + a_ref[1]) + dis * dis * h_ref[...]
    z = jnp.dot(m, w2_ref[...], preferred_element_type=jnp.float32)
    z = z + b2_ref[...]
    zmax = jnp.max(z, axis=1, keepdims=True)
    ez = jnp.exp(z - zmax)
    lse = jnp.log(jnp.sum(ez, axis=1, keepdims=True)) + zmax
    o_ref[...] = z - lse


def _fin(a2, h1, dis, w2, b2r):
    return pl.pallas_call(
        _fin_body,
        grid=(NPAD // _MB,),
        in_specs=[
            pl.BlockSpec((NC, _MB, L), lambda i: (0, i, 0)),
            pl.BlockSpec((_MB, L), lambda i: (i, 0)),
            pl.BlockSpec((_MB, L), lambda i: (i, 0)),
            pl.BlockSpec((L, 256), lambda i: (0, 0)),
            pl.BlockSpec((1, 256), lambda i: (0, 0)),
        ],
        out_specs=pl.BlockSpec((_MB, 256), lambda i: (i, 0)),
        out_shape=jax.ShapeDtypeStruct((N, 256), jnp.float32),
    )(a2, h1, dis, w2, b2r)


def kernel(x, edge_index, W1, b1, W2, b2):
    src = (jnp.zeros((EPAD,), jnp.int32).at[:E].set(edge_index[0])
           .reshape(ROWS, CHUNK))
    dst = (jnp.full((EPAD,), N, jnp.int32).at[:E].set(edge_index[1])
           .reshape(ROWS, CHUNK))
    b2r = b2.reshape(1, 256)

    hw1 = _mm1(x, W1)
    a1, dis = _mega1(hw1, src, dst)
    a2, h1 = _mega2(a1, hw1, dis, b1, src, dst)
    return _fin(a2, h1, dis, W2, b2r)


# speedup vs baseline: 43.5402x; 1.0680x over previous
import functools

import jax
import jax.numpy as jnp
from jax import lax
from jax.experimental import pallas as pl
from jax.experimental.pallas import tpu as pltpu
from jax.experimental.pallas import tpu_sc as plsc

N = 10000
NPAD = 10240
E = 160000
NC, NS, L = 2, 16, 16
NW = NC * NS
CHUNK = 128
EPT = 5120
EPAD = EPT * NW
NCHUNK = EPT // CHUNK
CPS = NCHUNK
ROWS = EPAD // CHUNK
EPSC = ROWS // NS
RPS = NPAD // NS
RING = 16
LOOK = 10

_mesh = plsc.VectorSubcoreMesh(core_axis_name="c", subcore_axis_name="s")
_sc_params = pltpu.CompilerParams(use_tc_tiling_on_sc=False,
                                  needs_layout_passes=False)


def _fill_rows(buf, nrows, vec):
    def body(i, _):
        buf[i, :] = vec
        return 0

    lax.fori_loop(0, nrows, body, 0)


def _rsqrt16(x):
    i = plsc.bitcast(x, jnp.int32)
    i = jnp.int32(0x5F3759DF) - (i >> 1)
    y = plsc.bitcast(i, jnp.float32)
    xh = x * 0.5
    y = y * (1.5 - xh * y * y)
    y = y * (1.5 - xh * y * y)
    y = y * (1.5 - xh * y * y)
    return y


def _agg_pipeline(sidx, didx, rows_v, tab_sh, acc_sh, sem_g, sem_s):
    def rows_at(j):
        return rows_v.at[pl.ds(lax.rem(j, RING) * CHUNK, CHUNK)]

    def gather(j):
        pltpu.async_copy(tab_sh.at[sidx.at[j]], rows_at(j), sem_g)

    def wait_gather(j):
        pltpu.make_async_copy(tab_sh.at[sidx.at[j]], rows_at(j), sem_g).wait()

    def scatter(j):
        pltpu.async_copy(rows_at(j), acc_sh.at[didx.at[j]], sem_s, add=True)

    def wait_scatter(j):
        pltpu.make_async_copy(rows_at(j), acc_sh.at[didx.at[j]], sem_s).wait()

    def prime(j, _):
        gather(j)
        return 0

    lax.fori_loop(0, LOOK, prime, 0)

    def step(j, _):
        @pl.when(j >= RING - LOOK)
        def _():
            wait_scatter(j - (RING - LOOK))

        @pl.when(j + LOOK < NCHUNK)
        def _():
            gather(j + LOOK)

        wait_gather(j)
        scatter(j)
        return 0

    lax.fori_loop(0, NCHUNK, step, 0)

    def drain(j, _):
        wait_scatter(j)
        return 0

    lax.fori_loop(NCHUNK - (RING - LOOK), NCHUNK, drain, 0)


@functools.partial(
    pl.kernel,
    out_type=(
        jax.ShapeDtypeStruct((NC, NPAD, L), jnp.float32),
        jax.ShapeDtypeStruct((NPAD, L), jnp.float32),
    ),
    mesh=_mesh,
    compiler_params=_sc_params,
    scratch_types=[
        pltpu.VMEM((EPSC, CHUNK), jnp.int32),
        pltpu.VMEM((NPAD,), jnp.float32),
        pltpu.VMEM((NS, RPS), jnp.float32),
        pltpu.VMEM((RPS,), jnp.float32),
        pltpu.VMEM((RPS, L), jnp.float32),
        pltpu.VMEM((RPS, L), jnp.float32),
        pltpu.VMEM((CPS, CHUNK), jnp.int32),
        pltpu.VMEM((CPS, CHUNK), jnp.int32),
        pltpu.VMEM((RING * CHUNK, L), jnp.float32),
        pltpu.VMEM_SHARED((NPAD, L), jnp.float32),
        pltpu.VMEM_SHARED((NPAD, L), jnp.float32),
        pltpu.VMEM_SHARED((NS, NPAD), jnp.float32),
        pltpu.SemaphoreType.DMA,
        pltpu.SemaphoreType.DMA,
    ],
)
def _mega1(hw_hbm, src_hbm, dst_hbm, a_out, dis_out,
           didx_deg, hist_v, hbuf, disv, rowbuf, disrow,
           sidx, didx, rows_v, tab_sh, acc_sh, hist_sh, sem_g, sem_s):
    cid = lax.axis_index("c")
    sid = lax.axis_index("s")
    wid = sid * NC + cid
    zero16 = jnp.zeros((L,), jnp.float32)
    one16 = jnp.ones((L,), jnp.float32)

    pltpu.async_copy(dst_hbm.at[pl.ds(sid * EPSC, EPSC)], didx_deg, sem_g)
    pltpu.async_copy(src_hbm.at[pl.ds(wid * CPS, CPS)], sidx, sem_g)
    pltpu.async_copy(dst_hbm.at[pl.ds(wid * CPS, CPS)], didx, sem_g)
    pltpu.async_copy(hw_hbm.at[pl.ds(sid * RPS, RPS)], rowbuf, sem_g)

    def hzero(i, _):
        hist_v[pl.ds(i * L, L)] = zero16
        return 0

    lax.fori_loop(0, NPAD // L, hzero, 0)
    pltpu.make_async_copy(dst_hbm.at[pl.ds(sid * EPSC, EPSC)], didx_deg,
                          sem_g).wait()
    pltpu.make_async_copy(src_hbm.at[pl.ds(wid * CPS, CPS)], sidx,
                          sem_g).wait()
    pltpu.make_async_copy(dst_hbm.at[pl.ds(wid * CPS, CPS)], didx,
                          sem_g).wait()
    pltpu.make_async_copy(hw_hbm.at[pl.ds(sid * RPS, RPS)], rowbuf,
                          sem_g).wait()

    def hrow(r, _):
        def hvec(k, _):
            iv = didx_deg[r, pl.ds(k * L, L)]
            plsc.addupdate_scatter(hist_v, [iv], one16)
            return 0

        lax.fori_loop(0, CHUNK // L, hvec, 0)
        return 0

    lax.fori_loop(0, EPSC, hrow, 0)
    pltpu.sync_copy(hist_v, hist_sh.at[sid])
    plsc.subcore_barrier()

    def hload(p, _):
        pltpu.async_copy(hist_sh.at[p].at[pl.ds(sid * RPS, RPS)], hbuf.at[p],
                         sem_g)
        return 0

    lax.fori_loop(0, NS, hload, 0)

    def hload_wait(p, _):
        pltpu.make_async_copy(hist_sh.at[p].at[pl.ds(sid * RPS, RPS)],
                              hbuf.at[p], sem_g).wait()
        return 0

    lax.fori_loop(0, NS, hload_wait, 0)

    def dvec(c, _):
        def hsum(p, acc):
            return acc + hbuf[p, pl.ds(c * L, L)]

        deg = lax.fori_loop(0, NS, hsum, one16)
        disv[pl.ds(c * L, L)] = _rsqrt16(deg)
        return 0

    lax.fori_loop(0, RPS // L, dvec, 0)

    def grow(c, _):
        dv = disv[pl.ds(c * L, L)]
        for k in range(L):
            i = c * L + k
            srow = jnp.full((L,), dv[k], jnp.float32)
            disrow[i, :] = srow
            rowbuf[i, :] = rowbuf[i, :] * srow
        return 0

    lax.fori_loop(0, RPS // L, grow, 0)

    @pl.when(cid == 0)
    def _():
        pltpu.sync_copy(disrow, dis_out.at[pl.ds(sid * RPS, RPS)])

    pltpu.sync_copy(rowbuf, tab_sh.at[pl.ds(sid * RPS, RPS)])
    _fill_rows(disrow, RPS, zero16)
    pltpu.sync_copy(disrow, acc_sh.at[pl.ds(sid * RPS, RPS)])
    plsc.subcore_barrier()

    _agg_pipeline(sidx, didx, rows_v, tab_sh, acc_sh, sem_g, sem_s)

    plsc.subcore_barrier()
    pltpu.sync_copy(acc_sh.at[pl.ds(sid * RPS, RPS)], rowbuf)
    pltpu.sync_copy(rowbuf, a_out.at[cid].at[pl.ds(sid * RPS, RPS)])


@functools.partial(
    pl.kernel,
    out_type=(
        jax.ShapeDtypeStruct((NC, NPAD, L), jnp.float32),
        jax.ShapeDtypeStruct((NPAD, L), jnp.float32),
    ),
    mesh=_mesh,
    compiler_params=_sc_params,
    scratch_types=[
        pltpu.VMEM((RPS, L), jnp.float32),
        pltpu.VMEM((RPS, L), jnp.float32),
        pltpu.VMEM((RPS, L), jnp.float32),
        pltpu.VMEM((RPS, L), jnp.float32),
        pltpu.VMEM((L,), jnp.float32),
        pltpu.VMEM((CPS, CHUNK), jnp.int32),
        pltpu.VMEM((CPS, CHUNK), jnp.int32),
        pltpu.VMEM((RING * CHUNK, L), jnp.float32),
        pltpu.VMEM_SHARED((NPAD, L), jnp.float32),
        pltpu.VMEM_SHARED((NPAD, L), jnp.float32),
        pltpu.SemaphoreType.DMA,
        pltpu.SemaphoreType.DMA,
    ],
)
def _mega2(a1_hbm, hw_hbm, dis_hbm, b1_hbm, src_hbm, dst_hbm,
           a_out, h1_out,
           abuf0, abuf1, rowbuf, disrow, b1v,
           sidx, didx, rows_v, tab_sh, acc_sh, sem_g, sem_s):
    cid = lax.axis_index("c")
    sid = lax.axis_index("s")
    wid = sid * NC + cid
    zero16 = jnp.zeros((L,), jnp.float32)
    sl = pl.ds(sid * RPS, RPS)

    pltpu.async_copy(a1_hbm.at[0].at[sl], abuf0, sem_g)
    pltpu.async_copy(a1_hbm.at[1].at[sl], abuf1, sem_g)
    pltpu.async_copy(hw_hbm.at[sl], rowbuf, sem_g)
    pltpu.async_copy(dis_hbm.at[sl], disrow, sem_g)
    pltpu.async_copy(b1_hbm, b1v, sem_g)
    pltpu.async_copy(src_hbm.at[pl.ds(wid * CPS, CPS)], sidx, sem_g)
    pltpu.async_copy(dst_hbm.at[pl.ds(wid * CPS, CPS)], didx, sem_g)
    pltpu.make_async_copy(a1_hbm.at[0].at[sl], abuf0, sem_g).wait()
    pltpu.make_async_copy(a1_hbm.at[1].at[sl], abuf1, sem_g).wait()
    pltpu.make_async_copy(hw_hbm.at[sl], rowbuf, sem_g).wait()
    pltpu.make_async_copy(dis_hbm.at[sl], disrow, sem_g).wait()
    pltpu.make_async_copy(b1_hbm, b1v, sem_g).wait()
    pltpu.make_async_copy(src_hbm.at[pl.ds(wid * CPS, CPS)], sidx,
                          sem_g).wait()
    pltpu.make_async_copy(dst_hbm.at[pl.ds(wid * CPS, CPS)], didx,
                          sem_g).wait()
    b1 = b1v[...]

    def hrow(i, _):
        d = disrow[i, :]
        agg = abuf0[i, :] + abuf1[i, :]
        h = jnp.maximum(d * agg + d * d * rowbuf[i, :] + b1, 0.0)
        abuf0[i, :] = h
        rowbuf[i, :] = h * d
        return 0

    lax.fori_loop(0, RPS, hrow, 0)

    @pl.when(cid == 0)
    def _():
        pltpu.sync_copy(abuf0, h1_out.at[sl])

    pltpu.sync_copy(rowbuf, tab_sh.at[sl])
    _fill_rows(abuf1, RPS, zero16)
    pltpu.sync_copy(abuf1, acc_sh.at[sl])
    plsc.subcore_barrier()

    _agg_pipeline(sidx, didx, rows_v, tab_sh, acc_sh, sem_g, sem_s)

    plsc.subcore_barrier()
    pltpu.sync_copy(acc_sh.at[sl], rowbuf)
    pltpu.sync_copy(rowbuf, a_out.at[cid].at[sl])


_MB = 1024


def _mm1_body(x_ref, w_ref, o_ref):
    o_ref[...] = jnp.dot(x_ref[...], w_ref[...],
                         preferred_element_type=jnp.float32)


def _mm1(x, w1):
    return pl.pallas_call(
        _mm1_body,
        grid=(NPAD // _MB,),
        in_specs=[
            pl.BlockSpec((_MB, 256), lambda i: (i, 0)),
            pl.BlockSpec((256, L), lambda i: (0, 0)),
        ],
        out_specs=pl.BlockSpec((_MB, L), lambda i: (i, 0)),
        out_shape=jax.ShapeDtypeStruct((NPAD, L), jnp.float32),
    )(x, w1)


def _fin_body(a_ref, h_ref, dis_ref, w2_ref, b2_ref, o_ref):
    dis = dis_ref[...]
    m = dis * (a_ref[0] + a_ref[1]) + dis * dis * h_ref[...]
    z = jnp.dot(m, w2_ref[...], preferred_element_type=jnp.float32)
    z = z + b2_ref[...]
    zmax = jnp.max(z, axis=1, keepdims=True)
    ez = jnp.exp(z - zmax)
    lse = jnp.log(jnp.sum(ez, axis=1, keepdims=True)) + zmax
    o_ref[...] = z - lse


def _fin(a2, h1, dis, w2, b2r):
    return pl.pallas_call(
        _fin_body,
        grid=(NPAD // _MB,),
        in_specs=[
            pl.BlockSpec((NC, _MB, L), lambda i: (0, i, 0)),
            pl.BlockSpec((_MB, L), lambda i: (i, 0)),
            pl.BlockSpec((_MB, L), lambda i: (i, 0)),
            pl.BlockSpec((L, 256), lambda i: (0, 0)),
            pl.BlockSpec((1, 256), lambda i: (0, 0)),
        ],
        out_specs=pl.BlockSpec((_MB, 256), lambda i: (i, 0)),
        out_shape=jax.ShapeDtypeStruct((N, 256), jnp.float32),
    )(a2, h1, dis, w2, b2r)


def kernel(x, edge_index, W1, b1, W2, b2):
    src = (jnp.zeros((EPAD,), jnp.int32).at[:E].set(edge_index[0])
           .reshape(ROWS, CHUNK))
    dst = (jnp.full((EPAD,), N, jnp.int32).at[:E].set(edge_index[1])
           .reshape(ROWS, CHUNK))
    b2r = b2.reshape(1, 256)

    hw1 = _mm1(x, W1)
    a1, dis = _mega1(hw1, src, dst)
    a2, h1 = _mega2(a1, hw1, dis, b1, src, dst)
    return _fin(a2, h1, dis, W2, b2r)


# speedup vs baseline: 43.8365x; 1.0068x over previous
import functools

import jax
import jax.numpy as jnp
from jax import lax
from jax.experimental import pallas as pl
from jax.experimental.pallas import tpu as pltpu
from jax.experimental.pallas import tpu_sc as plsc

N = 10000
NPAD = 10240
E = 160000
NC, NS, L = 2, 16, 16
NW = NC * NS
CHUNK = 128
EPT = 5120
EPAD = EPT * NW
NCHUNK = EPT // CHUNK
CPS = NCHUNK
ROWS = EPAD // CHUNK
EPSC = ROWS // NS
RPS = NPAD // NS
RING = 16
LOOK = 10

_mesh = plsc.VectorSubcoreMesh(core_axis_name="c", subcore_axis_name="s")
_sc_params = pltpu.CompilerParams(use_tc_tiling_on_sc=False,
                                  needs_layout_passes=False)


def _fill_rows(buf, nrows, vec):
    def body(i, _):
        buf[i, :] = vec
        return 0

    lax.fori_loop(0, nrows, body, 0)


def _rsqrt16(x):
    i = plsc.bitcast(x, jnp.int32)
    i = jnp.int32(0x5F3759DF) - (i >> 1)
    y = plsc.bitcast(i, jnp.float32)
    xh = x * 0.5
    y = y * (1.5 - xh * y * y)
    y = y * (1.5 - xh * y * y)
    y = y * (1.5 - xh * y * y)
    return y


def _agg_pipeline(sidx, didx, rows_v, tab_sh, acc_sh, sem_g, sem_s):
    def rows_at(j):
        return rows_v.at[pl.ds(lax.rem(j, RING) * CHUNK, CHUNK)]

    def gather(j):
        pltpu.async_copy(tab_sh.at[sidx.at[j]], rows_at(j), sem_g)

    def wait_gather(j):
        pltpu.make_async_copy(tab_sh.at[sidx.at[j]], rows_at(j), sem_g).wait()

    def scatter(j):
        pltpu.async_copy(rows_at(j), acc_sh.at[didx.at[j]], sem_s, add=True)

    def wait_scatter(j):
        pltpu.make_async_copy(rows_at(j), acc_sh.at[didx.at[j]], sem_s).wait()

    def prime(j, _):
        gather(j)
        return 0

    lax.fori_loop(0, LOOK, prime, 0)

    def step(j, _):
        @pl.when(j >= RING - LOOK)
        def _():
            wait_scatter(j - (RING - LOOK))

        @pl.when(j + LOOK < NCHUNK)
        def _():
            gather(j + LOOK)

        wait_gather(j)
        scatter(j)
        return 0

    lax.fori_loop(0, NCHUNK, step, 0)

    def drain(j, _):
        wait_scatter(j)
        return 0

    lax.fori_loop(NCHUNK - (RING - LOOK), NCHUNK, drain, 0)


@functools.partial(
    pl.kernel,
    out_type=(
        jax.ShapeDtypeStruct((NC, NPAD, L), jnp.float32),
        jax.ShapeDtypeStruct((NPAD, L), jnp.float32),
    ),
    mesh=_mesh,
    compiler_params=_sc_params,
    scratch_types=[
        pltpu.VMEM((EPSC, CHUNK), jnp.int32),
        pltpu.VMEM((RPS, L), jnp.float32),
        pltpu.VMEM((RPS, L), jnp.float32),
        pltpu.VMEM((RPS // L, L), jnp.float32),
        pltpu.VMEM((RPS, L), jnp.float32),
        pltpu.VMEM((RPS, L), jnp.float32),
        pltpu.VMEM((RPS, L), jnp.float32),
        pltpu.VMEM((CPS, CHUNK), jnp.int32),
        pltpu.VMEM((CPS, CHUNK), jnp.int32),
        pltpu.VMEM((RING * CHUNK, L), jnp.float32),
        pltpu.VMEM_SHARED((NPAD, L), jnp.float32),
        pltpu.VMEM_SHARED((NPAD, L), jnp.float32),
        pltpu.SemaphoreType.DMA,
        pltpu.SemaphoreType.DMA,
    ],
)
def _mega1(hw_hbm, src_hbm, dst_hbm, a_out, dis_out,
           didx_deg, hist_v, hbuf, disv, rowbuf, disrow, qbuf,
           sidx, didx, rows_v, tab_sh, acc_sh, sem_g, sem_s):
    cid = lax.axis_index("c")
    sid = lax.axis_index("s")
    wid = sid * NC + cid
    zero16 = jnp.zeros((L,), jnp.float32)
    one16 = jnp.ones((L,), jnp.float32)

    pltpu.async_copy(dst_hbm.at[pl.ds(sid * EPSC, EPSC)], didx_deg, sem_g)
    pltpu.async_copy(src_hbm.at[pl.ds(wid * CPS, CPS)], sidx, sem_g)
    pltpu.async_copy(dst_hbm.at[pl.ds(wid * CPS, CPS)], didx, sem_g)
    pltpu.async_copy(hw_hbm.at[pl.ds(sid * RPS, RPS)], rowbuf, sem_g)

    _fill_rows(hist_v, RPS, zero16)
    pltpu.make_async_copy(dst_hbm.at[pl.ds(sid * EPSC, EPSC)], didx_deg,
                          sem_g).wait()
    pltpu.make_async_copy(src_hbm.at[pl.ds(wid * CPS, CPS)], sidx,
                          sem_g).wait()
    pltpu.make_async_copy(dst_hbm.at[pl.ds(wid * CPS, CPS)], didx,
                          sem_g).wait()
    pltpu.make_async_copy(hw_hbm.at[pl.ds(sid * RPS, RPS)], rowbuf,
                          sem_g).wait()

    def hrow(r, _):
        def hvec(k, _):
            iv = didx_deg[r, pl.ds(k * L, L)]
            plsc.addupdate_scatter(hist_v, [iv >> 4, iv & 15], one16)
            return 0

        lax.fori_loop(0, CHUNK // L, hvec, 0)
        return 0

    lax.fori_loop(0, EPSC, hrow, 0)
    pltpu.sync_copy(hist_v, tab_sh.at[pl.ds(sid * RPS, RPS)])
    plsc.subcore_barrier()

    HR = RPS // L

    def hload(p, _):
        pltpu.async_copy(tab_sh.at[pl.ds(p * RPS + sid * HR, HR)],
                         hbuf.at[pl.ds(p * HR, HR)], sem_g)
        return 0

    lax.fori_loop(0, NS, hload, 0)

    def hload_wait(p, _):
        pltpu.make_async_copy(tab_sh.at[pl.ds(p * RPS + sid * HR, HR)],
                              hbuf.at[pl.ds(p * HR, HR)], sem_g).wait()
        return 0

    lax.fori_loop(0, NS, hload_wait, 0)
    plsc.subcore_barrier()

    def dvec(c, _):
        def hsum(p, acc):
            return acc + hbuf[p * HR + c, :]

        deg = lax.fori_loop(0, NS, hsum, one16)
        disv[c, :] = _rsqrt16(deg)
        return 0

    lax.fori_loop(0, RPS // L, dvec, 0)

    def grow(c, _):
        dv = disv[c, :]
        for k in range(L):
            i = c * L + k
            srow = jnp.full((L,), dv[k], jnp.float32)
            disrow[i, :] = srow
            rowbuf[i, :] = rowbuf[i, :] * srow
        return 0

    lax.fori_loop(0, RPS // L, grow, 0)

    @pl.when(cid == 0)
    def _():
        pltpu.sync_copy(disrow, dis_out.at[pl.ds(sid * RPS, RPS)])

    pltpu.sync_copy(rowbuf, tab_sh.at[pl.ds(sid * RPS, RPS)])
    _fill_rows(qbuf, RPS, zero16)
    pltpu.sync_copy(qbuf, acc_sh.at[pl.ds(sid * RPS, RPS)])
    plsc.subcore_barrier()

    _agg_pipeline(sidx, didx, rows_v, tab_sh, acc_sh, sem_g, sem_s)

    plsc.subcore_barrier()
    pltpu.sync_copy(acc_sh.at[pl.ds(sid * RPS, RPS)], qbuf)

    @pl.when(cid == 0)
    def _():
        def q0(i, _):
            qbuf[i, :] = disrow[i, :] * qbuf[i, :]
            return 0

        lax.fori_loop(0, RPS, q0, 0)

    @pl.when(cid == 1)
    def _():
        def q1(i, _):
            qbuf[i, :] = disrow[i, :] * (qbuf[i, :] + rowbuf[i, :])
            return 0

        lax.fori_loop(0, RPS, q1, 0)

    pltpu.sync_copy(qbuf, a_out.at[cid].at[pl.ds(sid * RPS, RPS)])


@functools.partial(
    pl.kernel,
    out_type=jax.ShapeDtypeStruct((NC, NPAD, L), jnp.float32),
    mesh=_mesh,
    compiler_params=_sc_params,
    scratch_types=[
        pltpu.VMEM((RPS, L), jnp.float32),
        pltpu.VMEM((RPS, L), jnp.float32),
        pltpu.VMEM((RPS, L), jnp.float32),
        pltpu.VMEM((RPS, L), jnp.float32),
        pltpu.VMEM((L,), jnp.float32),
        pltpu.VMEM((CPS, CHUNK), jnp.int32),
        pltpu.VMEM((CPS, CHUNK), jnp.int32),
        pltpu.VMEM((RING * CHUNK, L), jnp.float32),
        pltpu.VMEM_SHARED((NPAD, L), jnp.float32),
        pltpu.VMEM_SHARED((NPAD, L), jnp.float32),
        pltpu.SemaphoreType.DMA,
        pltpu.SemaphoreType.DMA,
    ],
)
def _mega2(a1_hbm, dis_hbm, b1_hbm, src_hbm, dst_hbm, a_out,
           abuf0, abuf1, rowbuf, disrow, b1v,
           sidx, didx, rows_v, tab_sh, acc_sh, sem_g, sem_s):
    cid = lax.axis_index("c")
    sid = lax.axis_index("s")
    wid = sid * NC + cid
    zero16 = jnp.zeros((L,), jnp.float32)
    sl = pl.ds(sid * RPS, RPS)

    pltpu.async_copy(a1_hbm.at[0].at[sl], abuf0, sem_g)
    pltpu.async_copy(a1_hbm.at[1].at[sl], abuf1, sem_g)
    pltpu.async_copy(dis_hbm.at[sl], disrow, sem_g)
    pltpu.async_copy(b1_hbm, b1v, sem_g)
    pltpu.async_copy(src_hbm.at[pl.ds(wid * CPS, CPS)], sidx, sem_g)
    pltpu.async_copy(dst_hbm.at[pl.ds(wid * CPS, CPS)], didx, sem_g)
    pltpu.make_async_copy(a1_hbm.at[0].at[sl], abuf0, sem_g).wait()
    pltpu.make_async_copy(a1_hbm.at[1].at[sl], abuf1, sem_g).wait()
    pltpu.make_async_copy(dis_hbm.at[sl], disrow, sem_g).wait()
    pltpu.make_async_copy(b1_hbm, b1v, sem_g).wait()
    pltpu.make_async_copy(src_hbm.at[pl.ds(wid * CPS, CPS)], sidx,
                          sem_g).wait()
    pltpu.make_async_copy(dst_hbm.at[pl.ds(wid * CPS, CPS)], didx,
                          sem_g).wait()
    b1 = b1v[...]

    def hrow(i, _):
        h = jnp.maximum(abuf0[i, :] + abuf1[i, :] + b1, 0.0)
        rowbuf[i, :] = h * disrow[i, :]
        return 0

    lax.fori_loop(0, RPS, hrow, 0)

    pltpu.sync_copy(rowbuf, tab_sh.at[sl])
    _fill_rows(abuf1, RPS, zero16)
    pltpu.sync_copy(abuf1, acc_sh.at[sl])
    plsc.subcore_barrier()

    _agg_pipeline(sidx, didx, rows_v, tab_sh, acc_sh, sem_g, sem_s)

    plsc.subcore_barrier()
    pltpu.sync_copy(acc_sh.at[sl], abuf1)

    @pl.when(cid == 0)
    def _():
        def q0(i, _):
            abuf1[i, :] = disrow[i, :] * abuf1[i, :]
            return 0

        lax.fori_loop(0, RPS, q0, 0)

    @pl.when(cid == 1)
    def _():
        def q1(i, _):
            abuf1[i, :] = disrow[i, :] * (abuf1[i, :] + rowbuf[i, :])
            return 0

        lax.fori_loop(0, RPS, q1, 0)

    pltpu.sync_copy(abuf1, a_out.at[cid].at[sl])


_MB = 1024


def _mm1_body(x_ref, w_ref, o_ref):
    o_ref[...] = jnp.dot(x_ref[...], w_ref[...],
                         preferred_element_type=jnp.float32)


def _mm1(x, w1):
    return pl.pallas_call(
        _mm1_body,
        grid=(NPAD // _MB,),
        in_specs=[
            pl.BlockSpec((_MB, 256), lambda i: (i, 0)),
            pl.BlockSpec((256, L), lambda i: (0, 0)),
        ],
        out_specs=pl.BlockSpec((_MB, L), lambda i: (i, 0)),
        out_shape=jax.ShapeDtypeStruct((NPAD, L), jnp.float32),
    )(x, w1)


def _fin_body(a_ref, w2_ref, b2_ref, o_ref):
    m = a_ref[0] + a_ref[1]
    z = jnp.dot(m, w2_ref[...], preferred_element_type=jnp.float32)
    z = z + b2_ref[...]
    zmax = jnp.max(z, axis=1, keepdims=True)
    ez = jnp.exp(z - zmax)
    lse = jnp.log(jnp.sum(ez, axis=1, keepdims=True)) + zmax
    o_ref[...] = z - lse


def _fin(a2, w2, b2r):
    return pl.pallas_call(
        _fin_body,
        grid=(NPAD // _MB,),
        in_specs=[
            pl.BlockSpec((NC, _MB, L), lambda i: (0, i, 0)),
            pl.BlockSpec((L, 256), lambda i: (0, 0)),
            pl.BlockSpec((1, 256), lambda i: (0, 0)),
        ],
        out_specs=pl.BlockSpec((_MB, 256), lambda i: (i, 0)),
        out_shape=jax.ShapeDtypeStruct((N, 256), jnp.float32),
    )(a2, w2, b2r)


def kernel(x, edge_index, W1, b1, W2, b2):
    src = (jnp.zeros((EPAD,), jnp.int32).at[:E].set(edge_index[0])
           .reshape(ROWS, CHUNK))
    dst = (jnp.full((EPAD,), N, jnp.int32).at[:E].set(edge_index[1])
           .reshape(ROWS, CHUNK))
    b2r = b2.reshape(1, 256)

    hw1 = _mm1(x, W1)
    q1, dis = _mega1(hw1, src, dst)
    q2 = _mega2(q1, dis, b1, src, dst)
    return _fin(q2, W2, b2r)
